# speedup vs baseline: 3.8387x; 3.8387x over previous
import functools

import jax
import jax.numpy as jnp
from jax import lax
from jax.experimental import pallas as pl
from jax.experimental.pallas import tpu as pltpu
from jax.experimental.pallas import tpu_sc as plsc

B, P, F = 4, 8192, 768
ROWS = B * P
NC, NS = 2, 16
NW = NC * NS
RPW = ROWS // NW
CHUNK = 64
NCHUNK = RPW // CHUNK

_mesh = plsc.VectorSubcoreMesh(core_axis_name="c", subcore_axis_name="s")


@functools.partial(
    pl.kernel,
    out_type=jax.ShapeDtypeStruct((ROWS, F), jnp.float32),
    mesh=_mesh,
    scratch_types=[
        pltpu.VMEM((RPW,), jnp.int32),
        pltpu.VMEM((2, CHUNK, F), jnp.float32),
        pltpu.SemaphoreType.DMA,
    ],
)
def _gather_kernel(x_hbm, idx_hbm, out_hbm, idx_v, rows_v, gsem):
    wid = lax.axis_index("s") * NC + lax.axis_index("c")
    base = wid * RPW

    pltpu.sync_copy(idx_hbm.at[pl.ds(base, RPW)], idx_v)

    pltpu.async_copy(x_hbm.at[idx_v.at[pl.ds(0, CHUNK)]], rows_v.at[0], gsem)
    pltpu.async_copy(x_hbm.at[idx_v.at[pl.ds(CHUNK, CHUNK)]], rows_v.at[1], gsem)

    @pl.loop(0, NCHUNK, step=2)
    def _(k):
        for b in range(2):
            c = k + b
            pltpu.make_async_copy(
                x_hbm.at[idx_v.at[pl.ds(0, CHUNK)]], rows_v.at[b], gsem
            ).wait()
            pltpu.sync_copy(
                rows_v.at[b], out_hbm.at[pl.ds(base + c * CHUNK, CHUNK)]
            )

            @pl.when(c + 2 < NCHUNK)
            def _():
                pltpu.async_copy(
                    x_hbm.at[idx_v.at[pl.ds((c + 2) * CHUNK, CHUNK)]],
                    rows_v.at[b],
                    gsem,
                )


def kernel(x, routing_map):
    x_flat = x.reshape(ROWS, F)
    idx_flat = (
        routing_map[None, :] + (P * jnp.arange(B, dtype=jnp.int32))[:, None]
    ).reshape(ROWS)
    out_flat = _gather_kernel(x_flat, idx_flat)
    return out_flat.reshape(B, P, F)
